# baseline (speedup 1.0000x reference)
import functools

import jax
import jax.numpy as jnp
from jax import lax
from jax.experimental import pallas as pl
from jax.experimental.pallas import tpu as pltpu
from jax.experimental.pallas import tpu_sc as plsc

N_NODES = 10000
D = 128
NC = 2
NS = 16
NW = NC * NS
CHUNK = 128
N_PAD = 10112
ROWS_PER_TILE = N_PAD // NS
ZROWS = 8


NB = 2


def _seg_body(p_hbm, src_hbm, dst_hbm, out_hbm,
              src_v, dst_v, buf0, buf1, zrow, acc, g0, g1, s0, s1):
    gsems = (g0, g1)
    ssems = (s0, s1)
    c = lax.axis_index("c")
    s = lax.axis_index("s")
    wid = s * NC + c
    nch2 = src_v.shape[0]

    for i in range(ZROWS):
        for j in range(D // 16):
            zrow[i, pl.ds(j * 16, 16)] = jnp.zeros((16,), jnp.float32)

    base = s * ROWS_PER_TILE

    def zloop(i, _):
        pltpu.sync_copy(zrow, acc.at[pl.ds(base + i * ZROWS, ZROWS)])
        return 0

    lax.fori_loop(0, ROWS_PER_TILE // ZROWS, zloop, 0)

    plsc.subcore_barrier()

    bufs = (buf0, buf1)

    def gissue(j, b):
        pltpu.async_copy(p_hbm.at[src_v.at[j]], bufs[b], gsems[b])

    def gwait(j, b):
        pltpu.make_async_copy(p_hbm.at[src_v.at[j]], bufs[b], gsems[b]).wait()

    def sissue(j, b):
        pltpu.async_copy(bufs[b], acc.at[dst_v.at[j]], ssems[b], add=True)

    def swait(j, b):
        pltpu.make_async_copy(bufs[b], acc.at[dst_v.at[j]], ssems[b]).wait()

    def eloop(g, _):
        j0 = g * NB
        for b in range(NB):
            gwait(j0 + b, b)
            sissue(j0 + b, b)
        for b in range(NB):
            swait(j0 + b, b)

            @pl.when(j0 + b + NB < nch2)
            def _():
                gissue(j0 + b + NB, b)
        return 0

    for h in range(2):
        pltpu.sync_copy(src_hbm.at[wid].at[pl.ds(h * nch2, nch2)], src_v)
        pltpu.sync_copy(dst_hbm.at[wid].at[pl.ds(h * nch2, nch2)], dst_v)
        for b in range(NB):
            gissue(b, b)
        lax.fori_loop(0, nch2 // NB, eloop, 0)

    plsc.subcore_barrier()

    rows = pl.ds(base, ROWS_PER_TILE)
    pltpu.sync_copy(acc.at[rows], out_hbm.at[c].at[rows])


def _make_segsum(nch):
    return pl.kernel(
        _seg_body,
        out_type=(jax.ShapeDtypeStruct((NC, N_PAD, D), jnp.float32),),
        mesh=plsc.VectorSubcoreMesh(core_axis_name="c", subcore_axis_name="s"),
        scratch_types=(
            pltpu.VMEM((nch // 2, CHUNK), jnp.int32),
            pltpu.VMEM((nch // 2, CHUNK), jnp.int32),
            pltpu.VMEM((CHUNK, D), jnp.float32),
            pltpu.VMEM((CHUNK, D), jnp.float32),
            pltpu.VMEM((ZROWS, D), jnp.float32),
            pltpu.VMEM_SHARED((N_PAD, D), jnp.float32),
        ) + (pltpu.SemaphoreType.DMA,) * (2 * NB),
        name="sc_segsum",
    )


def _deg_body(dst_hbm, deg_hbm, dst_v, ones_v, zdeg, deg):
    c = lax.axis_index("c")
    s = lax.axis_index("s")
    wid = s * NC + c
    nch = dst_v.shape[0]

    for i in range(ZROWS):
        for j in range(D // 16):
            zdeg[i, pl.ds(j * 16, 16)] = jnp.zeros((16,), jnp.float32)
    for i in range(CHUNK):
        for j in range(D // 16):
            ones_v[i, pl.ds(j * 16, 16)] = jnp.ones((16,), jnp.float32)

    base = s * ROWS_PER_TILE

    def zloop(i, _):
        pltpu.sync_copy(zdeg, deg.at[pl.ds(base + i * ZROWS, ZROWS)])
        return 0

    lax.fori_loop(0, ROWS_PER_TILE // ZROWS, zloop, 0)

    pltpu.sync_copy(dst_hbm.at[wid], dst_v)

    plsc.subcore_barrier()

    def floop(j, _):
        pltpu.sync_copy(ones_v, deg.at[dst_v.at[j]], add=True)
        return 0

    lax.fori_loop(0, nch, floop, 0)

    plsc.subcore_barrier()

    rows = pl.ds(base, ROWS_PER_TILE)
    pltpu.sync_copy(deg.at[rows], deg_hbm.at[c].at[rows])


def _make_deg(nch):
    return pl.kernel(
        _deg_body,
        out_type=(jax.ShapeDtypeStruct((NC, N_PAD, D), jnp.float32),),
        mesh=plsc.VectorSubcoreMesh(core_axis_name="c", subcore_axis_name="s"),
        scratch_types=(
            pltpu.VMEM((nch, CHUNK), jnp.int32),
            pltpu.VMEM((CHUNK, D), jnp.float32),
            pltpu.VMEM((ZROWS, D), jnp.float32),
            pltpu.VMEM_SHARED((N_PAD, D), jnp.float32),
        ),
        name="sc_degree",
    )


BLK = 1000


def _enc_kernel(x, we1, be1, we2, be2, wl0, wr0, b0, p0, r0):
    h = jax.nn.gelu(jnp.dot(x[...], we1[...], preferred_element_type=jnp.float32)
                    + be1[...])
    h = jnp.dot(h, we2[...], preferred_element_type=jnp.float32) + be2[...]
    p0[...] = jnp.dot(h, wl0[...], preferred_element_type=jnp.float32)
    r0[...] = jnp.dot(h, wr0[...], preferred_element_type=jnp.float32) + b0[...]


def _mid_kernel(sp, dp, r_prev, wl1, wr1, b1, p1, r1):
    ssum = sp[0] + sp[1]
    degv = dp[0, :, 0:1] + dp[1, :, 0:1]
    agg = ssum / jnp.clip(degv, 1.0, None)
    h = jax.nn.gelu(agg + r_prev[...])
    p1[...] = jnp.dot(h, wl1[...], preferred_element_type=jnp.float32)
    r1[...] = jnp.dot(h, wr1[...], preferred_element_type=jnp.float32) + b1[...]


def _dec_kernel(sp, dp, r_prev, wd1, bd1, wd2, bd2, out):
    ssum = sp[0] + sp[1]
    degv = dp[0, :, 0:1] + dp[1, :, 0:1]
    agg = ssum / jnp.clip(degv, 1.0, None)
    h = jax.nn.gelu(agg + r_prev[...])
    h = jax.nn.gelu(jnp.dot(h, wd1[...], preferred_element_type=jnp.float32)
                    + bd1[...])
    out[...] = jnp.dot(h, wd2[...], preferred_element_type=jnp.float32) + bd2[...]


def _row_spec():
    return pl.BlockSpec((BLK, D), lambda i: (i, 0))


def _full_spec(shape):
    n = len(shape)
    return pl.BlockSpec(shape, lambda i, _n=n: (0,) * _n)


def _part_spec():
    return pl.BlockSpec((NC, BLK, D), lambda i: (0, i, 0))


def _deg_spec():
    return pl.BlockSpec((NC, BLK, D), lambda i: (0, i, 0))


def kernel(x, edge_index, We1, be1, We2, be2, Wl0, Wr0, b0, Wl1, Wr1, b1,
           Wd1, bd1, Wd2, bd2):
    n = x.shape[0]
    e = edge_index.shape[1]

    ei = edge_index.astype(jnp.int32)
    epw = e // NW
    nch = (2 * NB) * pl.cdiv(epw, 2 * NB * CHUNK)
    epad = nch * CHUNK
    src = jnp.concatenate(
        [ei[0].reshape(NW, epw),
         jnp.zeros((NW, epad - epw), jnp.int32)], axis=1).reshape(NW, nch, CHUNK)
    dst = jnp.concatenate(
        [ei[1].reshape(NW, epw),
         jnp.full((NW, epad - epw), N_NODES, jnp.int32)], axis=1).reshape(NW, nch, CHUNK)

    b2 = lambda v: v.reshape(1, D)
    grid = n // BLK

    (deg16,) = _make_deg(nch)(dst)

    p0, r0 = pl.pallas_call(
        _enc_kernel,
        grid=(grid,),
        in_specs=[_row_spec()] + [_full_spec(s) for s in
                  [(D, D), (1, D), (D, D), (1, D), (D, D), (D, D), (1, D)]],
        out_specs=[_row_spec(), _row_spec()],
        out_shape=[jax.ShapeDtypeStruct((n, D), jnp.float32)] * 2,
    )(x, We1, b2(be1), We2, b2(be2), Wl0, Wr0, b2(b0))

    (s0,) = _make_segsum(nch)(p0, src, dst)

    p1, r1 = pl.pallas_call(
        _mid_kernel,
        grid=(grid,),
        in_specs=[_part_spec(), _deg_spec(), _row_spec(),
                  _full_spec((D, D)), _full_spec((D, D)), _full_spec((1, D))],
        out_specs=[_row_spec(), _row_spec()],
        out_shape=[jax.ShapeDtypeStruct((n, D), jnp.float32)] * 2,
    )(s0, deg16, r0, Wl1, Wr1, b2(b1))

    (s1,) = _make_segsum(nch)(p1, src, dst)

    out = pl.pallas_call(
        _dec_kernel,
        grid=(grid,),
        in_specs=[_part_spec(), _deg_spec(), _row_spec(),
                  _full_spec((D, D)), _full_spec((1, D)),
                  _full_spec((D, D)), _full_spec((1, D))],
        out_specs=_row_spec(),
        out_shape=jax.ShapeDtypeStruct((n, D), jnp.float32),
    )(s1, deg16, r1, Wd1, b2(bd1), Wd2, b2(bd2))

    return out


# speedup vs baseline: 1.0294x; 1.0294x over previous
import functools

import jax
import jax.numpy as jnp
from jax import lax
from jax.experimental import pallas as pl
from jax.experimental.pallas import tpu as pltpu
from jax.experimental.pallas import tpu_sc as plsc

N_NODES = 10000
D = 128
NC = 2
NS = 16
NW = NC * NS
CHUNK = 128
N_PAD = 10112
ROWS_PER_TILE = N_PAD // NS
ZROWS = 8


NB = 2


def _seg_body(p_hbm, src_hbm, dst_hbm, out_hbm,
              src_v, dst_v, buf0, buf1, zrow, acc, g0, g1, s0, s1):
    gsems = (g0, g1)
    ssems = (s0, s1)
    c = lax.axis_index("c")
    s = lax.axis_index("s")
    wid = s * NC + c
    nch2 = src_v.shape[0]

    for i in range(ZROWS):
        for j in range(D // 16):
            zrow[i, pl.ds(j * 16, 16)] = jnp.zeros((16,), jnp.float32)

    base = s * ROWS_PER_TILE

    def zloop(i, _):
        pltpu.sync_copy(zrow, acc.at[pl.ds(base + i * ZROWS, ZROWS)])
        return 0

    lax.fori_loop(0, ROWS_PER_TILE // ZROWS, zloop, 0)

    plsc.subcore_barrier()

    bufs = (buf0, buf1)

    def gissue(j, b):
        pltpu.async_copy(p_hbm.at[src_v.at[j]], bufs[b], gsems[b])

    def gwait(j, b):
        pltpu.make_async_copy(p_hbm.at[src_v.at[j]], bufs[b], gsems[b]).wait()

    def eloop(g, _):
        j0 = g * 2
        gwait(j0, 0)
        gissue(j0 + 1, 1)
        pltpu.sync_copy(buf0, acc.at[dst_v.at[j0]], add=True)
        gwait(j0 + 1, 1)

        @pl.when(g + 1 < nch2 // 2)
        def _():
            gissue(j0 + 2, 0)

        pltpu.sync_copy(buf1, acc.at[dst_v.at[j0 + 1]], add=True)
        return 0

    for h in range(2):
        pltpu.sync_copy(src_hbm.at[wid].at[pl.ds(h * nch2, nch2)], src_v)
        pltpu.sync_copy(dst_hbm.at[wid].at[pl.ds(h * nch2, nch2)], dst_v)
        gissue(0, 0)
        lax.fori_loop(0, nch2 // 2, eloop, 0)

    plsc.subcore_barrier()

    rows = pl.ds(base, ROWS_PER_TILE)
    pltpu.sync_copy(acc.at[rows], out_hbm.at[c].at[rows])


def _make_segsum(nch):
    return pl.kernel(
        _seg_body,
        out_type=(jax.ShapeDtypeStruct((NC, N_PAD, D), jnp.float32),),
        mesh=plsc.VectorSubcoreMesh(core_axis_name="c", subcore_axis_name="s"),
        scratch_types=(
            pltpu.VMEM((nch // 2, CHUNK), jnp.int32),
            pltpu.VMEM((nch // 2, CHUNK), jnp.int32),
            pltpu.VMEM((CHUNK, D), jnp.float32),
            pltpu.VMEM((CHUNK, D), jnp.float32),
            pltpu.VMEM((ZROWS, D), jnp.float32),
            pltpu.VMEM_SHARED((N_PAD, D), jnp.float32),
        ) + (pltpu.SemaphoreType.DMA,) * (2 * NB),
        name="sc_segsum",
    )


def _deg_body(dst_hbm, deg_hbm, dst_v, ones_v, zdeg, deg):
    c = lax.axis_index("c")
    s = lax.axis_index("s")
    wid = s * NC + c
    nch = dst_v.shape[0]

    for i in range(ZROWS):
        for j in range(D // 16):
            zdeg[i, pl.ds(j * 16, 16)] = jnp.zeros((16,), jnp.float32)
    for i in range(CHUNK):
        for j in range(D // 16):
            ones_v[i, pl.ds(j * 16, 16)] = jnp.ones((16,), jnp.float32)

    base = s * ROWS_PER_TILE

    def zloop(i, _):
        pltpu.sync_copy(zdeg, deg.at[pl.ds(base + i * ZROWS, ZROWS)])
        return 0

    lax.fori_loop(0, ROWS_PER_TILE // ZROWS, zloop, 0)

    pltpu.sync_copy(dst_hbm.at[wid], dst_v)

    plsc.subcore_barrier()

    def floop(j, _):
        pltpu.sync_copy(ones_v, deg.at[dst_v.at[j]], add=True)
        return 0

    lax.fori_loop(0, nch, floop, 0)

    plsc.subcore_barrier()

    rows = pl.ds(base, ROWS_PER_TILE)
    pltpu.sync_copy(deg.at[rows], deg_hbm.at[c].at[rows])


def _make_deg(nch):
    return pl.kernel(
        _deg_body,
        out_type=(jax.ShapeDtypeStruct((NC, N_PAD, D), jnp.float32),),
        mesh=plsc.VectorSubcoreMesh(core_axis_name="c", subcore_axis_name="s"),
        scratch_types=(
            pltpu.VMEM((nch, CHUNK), jnp.int32),
            pltpu.VMEM((CHUNK, D), jnp.float32),
            pltpu.VMEM((ZROWS, D), jnp.float32),
            pltpu.VMEM_SHARED((N_PAD, D), jnp.float32),
        ),
        name="sc_degree",
    )


BLK = 1000


def _enc_kernel(x, we1, be1, we2, be2, wl0, wr0, b0, p0, r0):
    h = jax.nn.gelu(jnp.dot(x[...], we1[...], preferred_element_type=jnp.float32)
                    + be1[...])
    h = jnp.dot(h, we2[...], preferred_element_type=jnp.float32) + be2[...]
    p0[...] = jnp.dot(h, wl0[...], preferred_element_type=jnp.float32)
    r0[...] = jnp.dot(h, wr0[...], preferred_element_type=jnp.float32) + b0[...]


def _mid_kernel(sp, dp, r_prev, wl1, wr1, b1, p1, r1):
    ssum = sp[0] + sp[1]
    degv = dp[0, :, 0:1] + dp[1, :, 0:1]
    agg = ssum / jnp.clip(degv, 1.0, None)
    h = jax.nn.gelu(agg + r_prev[...])
    p1[...] = jnp.dot(h, wl1[...], preferred_element_type=jnp.float32)
    r1[...] = jnp.dot(h, wr1[...], preferred_element_type=jnp.float32) + b1[...]


def _dec_kernel(sp, dp, r_prev, wd1, bd1, wd2, bd2, out):
    ssum = sp[0] + sp[1]
    degv = dp[0, :, 0:1] + dp[1, :, 0:1]
    agg = ssum / jnp.clip(degv, 1.0, None)
    h = jax.nn.gelu(agg + r_prev[...])
    h = jax.nn.gelu(jnp.dot(h, wd1[...], preferred_element_type=jnp.float32)
                    + bd1[...])
    out[...] = jnp.dot(h, wd2[...], preferred_element_type=jnp.float32) + bd2[...]


def _row_spec():
    return pl.BlockSpec((BLK, D), lambda i: (i, 0))


def _full_spec(shape):
    n = len(shape)
    return pl.BlockSpec(shape, lambda i, _n=n: (0,) * _n)


def _part_spec():
    return pl.BlockSpec((NC, BLK, D), lambda i: (0, i, 0))


def _deg_spec():
    return pl.BlockSpec((NC, BLK, D), lambda i: (0, i, 0))


def kernel(x, edge_index, We1, be1, We2, be2, Wl0, Wr0, b0, Wl1, Wr1, b1,
           Wd1, bd1, Wd2, bd2):
    n = x.shape[0]
    e = edge_index.shape[1]

    ei = edge_index.astype(jnp.int32)
    epw = e // NW
    nch = (2 * NB) * pl.cdiv(epw, 2 * NB * CHUNK)
    epad = nch * CHUNK
    src = jnp.concatenate(
        [ei[0].reshape(NW, epw),
         jnp.zeros((NW, epad - epw), jnp.int32)], axis=1).reshape(NW, nch, CHUNK)
    dst = jnp.concatenate(
        [ei[1].reshape(NW, epw),
         jnp.full((NW, epad - epw), N_NODES, jnp.int32)], axis=1).reshape(NW, nch, CHUNK)

    b2 = lambda v: v.reshape(1, D)
    grid = n // BLK

    (deg16,) = _make_deg(nch)(dst)

    p0, r0 = pl.pallas_call(
        _enc_kernel,
        grid=(grid,),
        in_specs=[_row_spec()] + [_full_spec(s) for s in
                  [(D, D), (1, D), (D, D), (1, D), (D, D), (D, D), (1, D)]],
        out_specs=[_row_spec(), _row_spec()],
        out_shape=[jax.ShapeDtypeStruct((n, D), jnp.float32)] * 2,
    )(x, We1, b2(be1), We2, b2(be2), Wl0, Wr0, b2(b0))

    (s0,) = _make_segsum(nch)(p0, src, dst)

    p1, r1 = pl.pallas_call(
        _mid_kernel,
        grid=(grid,),
        in_specs=[_part_spec(), _deg_spec(), _row_spec(),
                  _full_spec((D, D)), _full_spec((D, D)), _full_spec((1, D))],
        out_specs=[_row_spec(), _row_spec()],
        out_shape=[jax.ShapeDtypeStruct((n, D), jnp.float32)] * 2,
    )(s0, deg16, r0, Wl1, Wr1, b2(b1))

    (s1,) = _make_segsum(nch)(p1, src, dst)

    out = pl.pallas_call(
        _dec_kernel,
        grid=(grid,),
        in_specs=[_part_spec(), _deg_spec(), _row_spec(),
                  _full_spec((D, D)), _full_spec((1, D)),
                  _full_spec((D, D)), _full_spec((1, D))],
        out_specs=_row_spec(),
        out_shape=jax.ShapeDtypeStruct((n, D), jnp.float32),
    )(s1, deg16, r1, Wd1, b2(bd1), Wd2, b2(bd2))

    return out
